# Initial kernel scaffold; baseline (speedup 1.0000x reference)
import jax
import jax.numpy as jnp
from jax.experimental import pallas as pl


def kernel(x, edge_index, edge_attr, batch, Wl, bl, Wr, br, We, att, bias, bn_g, bn_b, bn_m, bn_v, W1, b1, W2, b2, W3, b3):
    raise NotImplementedError("write your pallas kernel here")


# speedup vs baseline: 39.4205x; 39.4205x over previous
import functools

import jax
import jax.numpy as jnp
from jax import lax
from jax.experimental import pallas as pl
from jax.experimental.pallas import tpu as pltpu
from jax.experimental.pallas import tpu_sc as plsc

F32 = jnp.float32
I32 = jnp.int32

N = 10000
F_IN = 256
E = 160000
H = 4
C = 32
HC = 128
G = 556
G_PAD = 560
NEG = 0.01
EPS = 1e-5

NW = 32
CH = 64
ET = E + N
K_CH = 84
PER_W = CH * K_CH
ET_PAD = NW * PER_W
NP = 10240
ROWS_PER_TILE = NP // 16
NB = 10
BN = N // NB


def _proj_body(xb, wl, wr, bl, br, oxl, oxr):
    xv = xb[...]
    dn = (((1,), (1,)), ((), ()))
    oxl[...] = lax.dot_general(xv, wl[...], dn, preferred_element_type=F32) + bl[...]
    oxr[...] = lax.dot_general(xv, wr[...], dn, preferred_element_type=F32) + br[...]


def _ea_body(eab, oea):
    v = eab[...]
    ri = lax.broadcasted_iota(I32, v.shape, 0)
    ci = lax.broadcasted_iota(I32, v.shape, 1)
    gi = ri * v.shape[1] + ci
    m = gi < E
    mean = jnp.sum(jnp.where(m, v, 0.0)) / E
    oea[...] = jnp.where(m, v, mean)


def _edge_kernel(xl_hbm, xr_hbm, src_hbm, dst_hbm, ea_hbm, we_hbm, att_hbm,
                 out_u, out_s,
                 src_v, dst_v, ea_v, a_v, b_v, msg_v, sv_v, we_v, att_v,
                 row_v, u_sh, s_sh, sem_a, sem_b):
    c = lax.axis_index("c")
    s = lax.axis_index("s")
    w = s * 2 + c
    lane16 = lax.iota(I32, 16)

    def set_row_ids(base):
        for k in range(CH // 16):
            row_v[pl.ds(16 * k, 16)] = base + 16 * k + lane16

    pltpu.sync_copy(we_hbm, we_v)
    pltpu.sync_copy(att_hbm, att_v)
    we_r = [we_v[pl.ds(16 * k, 16)] for k in range(8)]
    att_r = [att_v[pl.ds(16 * k, 16)] for k in range(8)]
    lane = lax.iota(I32, 16)
    zero16 = jnp.zeros((16,), F32)

    def _zrow(r, _):
        for k in range(HC // 16):
            msg_v[r, pl.ds(16 * k, 16)] = zero16
        sv_v[r, :] = zero16
        return 0
    lax.fori_loop(0, CH, _zrow, 0)
    tb = s * ROWS_PER_TILE
    for i in range(ROWS_PER_TILE // CH):
        set_row_ids(tb + i * CH)
        pltpu.sync_copy(msg_v, u_sh.at[row_v])
        pltpu.sync_copy(sv_v, s_sh.at[row_v])
    plsc.subcore_barrier()

    ebase = w * PER_W

    def chunk_body(t, _):
        eb = ebase + t * CH
        pltpu.sync_copy(src_hbm.at[pl.ds(eb, CH)], src_v)
        pltpu.sync_copy(dst_hbm.at[pl.ds(eb, CH)], dst_v)
        pltpu.sync_copy(ea_hbm.at[pl.ds(eb, CH)], ea_v)
        ga = pltpu.async_copy(xl_hbm.at[src_v], a_v, sem_a)
        gb = pltpu.async_copy(xr_hbm.at[dst_v], b_v, sem_b)
        ga.wait()
        gb.wait()

        def group_body(g, _):
            ea_grp = ea_v[pl.ds(g * 16, 16)]
            for j in range(16):
                e = g * 16 + j
                ea_b = jnp.full((16,), ea_grp[j], F32)
                a = [a_v[e, pl.ds(16 * k, 16)] for k in range(8)]
                z = jnp.full((16,), -1e30, F32)
                for h in range(H):
                    q = None
                    for k in (2 * h, 2 * h + 1):
                        b = b_v[e, pl.ds(16 * k, 16)]
                        t1 = a[k] + b + ea_b * we_r[k]
                        t1 = jnp.maximum(t1, NEG * t1)
                        p = t1 * att_r[k]
                        q = p if q is None else q + p
                    ah = jnp.sum(q)
                    z = jnp.where(lane == h, ah, z)
                pen = jnp.where(eb + e < ET, 0.0, -1e30).astype(F32)
                svec = jnp.exp(z + pen)
                sv_v[e, :] = svec
                sb = [jnp.full((16,), svec[h], F32) for h in range(H)]
                for k in range(8):
                    msg_v[e, pl.ds(16 * k, 16)] = a[k] * sb[k // 2]
            return 0

        lax.fori_loop(0, CH // 16, group_body, 0)
        pltpu.sync_copy(msg_v, u_sh.at[dst_v], add=True)
        pltpu.sync_copy(sv_v, s_sh.at[dst_v], add=True)
        return 0

    lax.fori_loop(0, K_CH, chunk_body, 0)
    plsc.subcore_barrier()

    def dump_body(i, _):
        r = tb + i * CH
        set_row_ids(r)
        pltpu.sync_copy(u_sh.at[row_v], msg_v)
        pltpu.sync_copy(s_sh.at[row_v], sv_v)
        pltpu.sync_copy(msg_v, out_u.at[pl.ds(c * NP + r, CH)])
        pltpu.sync_copy(sv_v, out_s.at[pl.ds(c * NP + r, CH)])
        return 0

    lax.fori_loop(0, ROWS_PER_TILE // CH, dump_body, 0)


def _post_body(u0, u1, s0, s1, bt, bias, bng, bnb, bnm, bnv,
               w1, b1, w2, b2, w3p, b3r, out, acc_h, acc_c):
    i = pl.program_id(0)

    @pl.when(i == 0)
    def _():
        acc_h[...] = jnp.zeros_like(acc_h)
        acc_c[...] = jnp.zeros_like(acc_c)

    u = u0[...] + u1[...]
    sp = s0[...] + s1[...]
    ri = lax.broadcasted_iota(I32, (16, HC), 0)
    ci = lax.broadcasted_iota(I32, (16, HC), 1)
    sel = (ri == ci // C).astype(F32)
    dsum = jnp.dot(sp, sel, preferred_element_type=F32)
    h = u / (dsum + 1e-16) + bias[...]
    scale = bng[...] * lax.rsqrt(bnv[...] + EPS)
    h = (h - bnm[...]) * scale + bnb[...]
    h = jnp.maximum(h, NEG * h)

    b = bt[...].reshape(1, BN)
    g0 = lax.broadcasted_iota(I32, (G_PAD, BN), 0)
    ohf = (g0 == b).astype(F32)
    acc_h[...] += jnp.dot(ohf, h, preferred_element_type=F32)
    acc_c[...] += jnp.dot(ohf, jnp.ones((BN, HC), F32),
                          preferred_element_type=F32)

    @pl.when(i == NB - 1)
    def _():
        dn = (((1,), (1,)), ((), ()))
        pooled = acc_h[...] / jnp.maximum(acc_c[...], 1.0)
        z1 = lax.dot_general(pooled, w1[...], dn, preferred_element_type=F32) + b1[...]
        z1 = jnp.maximum(z1, NEG * z1)
        z2 = lax.dot_general(z1, w2[...], dn, preferred_element_type=F32) + b2[...]
        z2 = jnp.maximum(z2, NEG * z2)
        out[...] = lax.dot_general(z2, w3p[...], dn, preferred_element_type=F32) + b3r[...]


def kernel(x, edge_index, edge_attr, batch, Wl, bl, Wr, br, We, att, bias,
           bn_g, bn_b, bn_m, bn_v, W1, b1, W2, b2, W3, b3):
    xl, xr = pl.pallas_call(
        _proj_body,
        grid=(NB,),
        in_specs=[
            pl.BlockSpec((BN, F_IN), lambda i: (i, 0)),
            pl.BlockSpec((HC, F_IN), lambda i: (0, 0)),
            pl.BlockSpec((HC, F_IN), lambda i: (0, 0)),
            pl.BlockSpec((1, HC), lambda i: (0, 0)),
            pl.BlockSpec((1, HC), lambda i: (0, 0)),
        ],
        out_specs=[
            pl.BlockSpec((BN, HC), lambda i: (i, 0)),
            pl.BlockSpec((BN, HC), lambda i: (i, 0)),
        ],
        out_shape=[
            jax.ShapeDtypeStruct((N, HC), F32),
            jax.ShapeDtypeStruct((N, HC), F32),
        ],
    )(x, Wl, Wr, bl.reshape(1, HC), br.reshape(1, HC))

    ea_rows = ET_PAD // 128
    ea_pad = jnp.pad(edge_attr[:, 0], (0, ET_PAD - E)).reshape(ea_rows, 128)
    ea_full = pl.pallas_call(
        _ea_body,
        in_specs=[pl.BlockSpec((ea_rows, 128), lambda: (0, 0))],
        out_specs=pl.BlockSpec((ea_rows, 128), lambda: (0, 0)),
        out_shape=jax.ShapeDtypeStruct((ea_rows, 128), F32),
    )(ea_pad).reshape(ET_PAD)

    loop = jnp.arange(N, dtype=I32)
    padz = jnp.zeros((ET_PAD - ET,), I32)
    src_full = jnp.concatenate([edge_index[0], loop, padz])
    dst_full = jnp.concatenate([edge_index[1], loop, padz])

    mesh = plsc.VectorSubcoreMesh(core_axis_name="c", subcore_axis_name="s")
    edge_call = functools.partial(
        pl.kernel,
        out_type=(
            jax.ShapeDtypeStruct((2 * NP, HC), F32),
            jax.ShapeDtypeStruct((2 * NP, 16), F32),
        ),
        mesh=mesh,
        scratch_types=[
            pltpu.VMEM((CH,), I32),
            pltpu.VMEM((CH,), I32),
            pltpu.VMEM((CH,), F32),
            pltpu.VMEM((CH, HC), F32),
            pltpu.VMEM((CH, HC), F32),
            pltpu.VMEM((CH, HC), F32),
            pltpu.VMEM((CH, 16), F32),
            pltpu.VMEM((HC,), F32),
            pltpu.VMEM((HC,), F32),
            pltpu.VMEM((CH,), I32),
            pltpu.VMEM_SHARED((NP, HC), F32),
            pltpu.VMEM_SHARED((NP, 16), F32),
            pltpu.SemaphoreType.DMA,
            pltpu.SemaphoreType.DMA,
        ],
        compiler_params=pltpu.CompilerParams(needs_layout_passes=False),
    )(_edge_kernel)
    out_u, out_s = edge_call(xl, xr, src_full, dst_full, ea_full,
                             We.reshape(HC), att.reshape(HC))

    w3p = jnp.zeros((HC, 32), F32).at[0].set(W3[0])
    b3r = jnp.broadcast_to(b3.reshape(1, 1), (1, HC))
    full = lambda shape: pl.BlockSpec(shape, lambda i: tuple(0 for _ in shape))
    res = pl.pallas_call(
        _post_body,
        grid=(NB,),
        in_specs=[
            pl.BlockSpec((BN, HC), lambda i: (i, 0)),
            pl.BlockSpec((BN, HC), lambda i: (i, 0)),
            pl.BlockSpec((BN, 16), lambda i: (i, 0)),
            pl.BlockSpec((BN, 16), lambda i: (i, 0)),
            pl.BlockSpec((1, 1, BN), lambda i: (i, 0, 0)),
            full((1, HC)), full((1, HC)), full((1, HC)), full((1, HC)),
            full((1, HC)),
            full((64, HC)), full((1, 64)),
            full((32, 64)), full((1, 32)),
            full((HC, 32)), full((1, HC)),
        ],
        out_specs=pl.BlockSpec((G_PAD, HC), lambda i: (0, 0)),
        out_shape=jax.ShapeDtypeStruct((G_PAD, HC), F32),
        scratch_shapes=[
            pltpu.VMEM((G_PAD, HC), F32),
            pltpu.VMEM((G_PAD, HC), F32),
        ],
    )(out_u[:NP], out_u[NP:], out_s[:NP], out_s[NP:],
      batch.reshape(NB, 1, BN),
      bias.reshape(1, HC), bn_g.reshape(1, HC), bn_b.reshape(1, HC),
      bn_m.reshape(1, HC), bn_v.reshape(1, HC),
      W1, b1.reshape(1, 64), W2, b2.reshape(1, 32), w3p, b3r)
    return res[:G, 0]
